# baseline (speedup 1.0000x reference)
import functools

import jax
import jax.numpy as jnp
from jax.experimental import pallas as pl
from jax.experimental.pallas import tpu as pltpu


def _ggnn_kernel(hadj_ref, edge_ref, w_ref, out_ref, *,
                 num_layers, n_nodes, fdim, edim):
    f32 = jnp.float32
    F = fdim
    N = n_nodes
    E = edim
    L = num_layers
    h0 = hadj_ref[0, :N, :].astype(f32)
    adj = hadj_ref[0, N:, :].astype(f32)

    deg = jnp.sum(adj, axis=1, keepdims=True)

    e2 = edge_ref[0].reshape(N * N, E)
    adj8 = jnp.concatenate([adj] * 8, axis=1)
    row_i = jax.lax.broadcasted_iota(jnp.int32, (N, 8 * N), 0)
    col_b = jax.lax.broadcasted_iota(jnp.int32, (N, 8 * N), 1) // N
    q = row_i - col_b
    ew0 = jnp.zeros((N, E), f32)
    ew1 = jnp.zeros((N, E), f32)
    for t in range(N // 8):
        lhs = jnp.where(q == 8 * t, adj8, 0.0)
        part = jnp.dot(lhs, e2[8 * N * t:8 * N * (t + 1), :],
                       preferred_element_type=f32)
        if t % 2 == 0:
            ew0 = ew0 + part
        else:
            ew1 = ew1 + part
    ew = ew0 + ew1

    wh5 = lambda l: w_ref[l * F:(l + 1) * F, :5 * F]
    whm3 = lambda l: w_ref[L * F + l * F:L * F + (l + 1) * F, :3 * F]
    we4 = w_ref[2 * L * F:2 * L * F + E, :L * F]
    wread = w_ref[2 * L * F + E:2 * L * F + E + 2 * F, :2 * F]
    bias0 = 2 * L * F + E + 2 * F
    mb = lambda l: w_ref[bias0 + l:bias0 + l + 1, :F]
    brz = lambda l: w_ref[bias0 + L + l:bias0 + L + l + 1, :2 * F]
    bin_ = lambda l: w_ref[bias0 + 2 * L + l:bias0 + 2 * L + l + 1, :F]
    bhn = lambda l: w_ref[bias0 + 3 * L + l:bias0 + 3 * L + l + 1, :F]
    bl1 = w_ref[bias0 + 4 * L:bias0 + 4 * L + 1, :F]
    bl2 = w_ref[bias0 + 4 * L + 1:bias0 + 4 * L + 2, :F]

    ec_all = jnp.dot(ew, we4, preferred_element_type=f32)
    inv_n = 1.0 / float(N)

    h = h0
    for l in range(L):
        ph = jnp.dot(h, wh5(l), preferred_element_type=f32)
        agg = jnp.dot(adj, ph[:, :F], preferred_element_type=f32)
        m = (agg + ec_all[:, l * F:(l + 1) * F]
             + deg * (ph[:, F:2 * F] + mb(l))) * inv_n

        pm = jnp.dot(m, whm3(l), preferred_element_type=f32)

        rz = jax.nn.sigmoid(ph[:, 2 * F:4 * F] + pm[:, :2 * F] + brz(l))
        r = rz[:, :F]
        z = rz[:, F:]
        n = jnp.tanh(ph[:, 4 * F:] + bin_(l)
                     + r * (pm[:, 2 * F:] + bhn(l)))
        h = jnp.maximum((1.0 - z) * n + z * m, 0.0)

    gl = jnp.dot(jnp.concatenate([h, h0], axis=1), wread,
                 preferred_element_type=f32)
    g = jax.nn.sigmoid(gl[:, :F] + bl1)
    hl2 = gl[:, F:] + bl2
    r_out = jnp.mean(g * hl2, axis=0, keepdims=True)
    out_ref[...] = jnp.maximum(r_out, 0.0).reshape(out_ref.shape).astype(out_ref.dtype)


def _pack(layers, L1, bL1, L2, bL2, fdim, edim):
    F, E = fdim, edim
    L = len(layers)
    W5 = 5 * F

    def pad(x):
        return jnp.pad(x, ((0, 0), (0, W5 - x.shape[1])))

    rows = []
    for lp in layers:
        rows.append(jnp.concatenate(
            [lp["W"][:, :F].T, lp["W"][:, F + E:].T,
             lp["Wih"][0:F].T, lp["Wih"][F:2 * F].T, lp["Wih"][2 * F:].T],
            axis=1))
    for lp in layers:
        rows.append(pad(jnp.concatenate(
            [lp["Whh"][0:F].T, lp["Whh"][F:2 * F].T, lp["Whh"][2 * F:].T],
            axis=1)))
    rows.append(pad(jnp.concatenate(
        [lp["W"][:, F:F + E].T for lp in layers], axis=1)))
    rows.append(pad(jnp.concatenate([
        jnp.concatenate([L1[:, :F].T, L2.T], axis=1),
        jnp.concatenate([L1[:, F:].T, jnp.zeros((F, F), jnp.float32)], axis=1),
    ], axis=0)))
    bias_rows = []
    for lp in layers:
        bias_rows.append(pad(lp["Wb"].reshape(1, F)))
    for lp in layers:
        bias_rows.append(pad((lp["bih"][:2 * F] + lp["bhh"][:2 * F]).reshape(1, 2 * F)))
    for lp in layers:
        bias_rows.append(pad(lp["bih"][2 * F:].reshape(1, F)))
    for lp in layers:
        bias_rows.append(pad(lp["bhh"][2 * F:].reshape(1, F)))
    bias_rows.append(pad(bL1.reshape(1, F)))
    bias_rows.append(pad(bL2.reshape(1, F)))
    big = jnp.concatenate(rows + bias_rows, axis=0)
    r = big.shape[0]
    big = jnp.pad(big, ((0, (-r) % 8), (0, 0)))
    return big


def kernel(h, edge, adj,
           ly0_W, ly0_Wb, ly0_Wih, ly0_Whh, ly0_bih, ly0_bhh,
           ly1_W, ly1_Wb, ly1_Wih, ly1_Whh, ly1_bih, ly1_bhh,
           ly2_W, ly2_Wb, ly2_Wih, ly2_Whh, ly2_bih, ly2_bhh,
           ly3_W, ly3_Wb, ly3_Wih, ly3_Whh, ly3_bih, ly3_bhh,
           L1, bL1, L2, bL2):
    B, N, F = h.shape
    E = edge.shape[-1]
    layers = [
        {"W": ly0_W, "Wb": ly0_Wb, "Wih": ly0_Wih, "Whh": ly0_Whh,
         "bih": ly0_bih, "bhh": ly0_bhh},
        {"W": ly1_W, "Wb": ly1_Wb, "Wih": ly1_Wih, "Whh": ly1_Whh,
         "bih": ly1_bih, "bhh": ly1_bhh},
        {"W": ly2_W, "Wb": ly2_Wb, "Wih": ly2_Wih, "Whh": ly2_Whh,
         "bih": ly2_bih, "bhh": ly2_bhh},
        {"W": ly3_W, "Wb": ly3_Wb, "Wih": ly3_Wih, "Whh": ly3_Whh,
         "bih": ly3_bih, "bhh": ly3_bhh},
    ]
    L = len(layers)
    big = _pack(layers, L1, bL1, L2, bL2, F, E)
    hadj = jnp.concatenate([h, adj], axis=1)

    body = functools.partial(_ggnn_kernel, num_layers=L, n_nodes=N,
                             fdim=F, edim=E)

    flops_per_b = (L * (2 * N * F * 5 * F + 2 * N * N * F + 2 * N * F * 3 * F
                        + 20 * N * F)
                   + 2 * N * N * E + 2 * N * E * L * F + 2 * N * 2 * F * 2 * F
                   + 10 * N * F)
    transc_per_b = L * 3 * N * F + N * F
    in_bytes = (hadj.size + edge.size + big.size) * 4
    cost = pl.CostEstimate(flops=int(B * flops_per_b),
                           transcendentals=int(B * transc_per_b),
                           bytes_accessed=int(in_bytes + B * F * 4))

    R = big.shape[0]
    out = pl.pallas_call(
        body,
        out_shape=jax.ShapeDtypeStruct((B, 1, F), h.dtype),
        grid_spec=pltpu.PrefetchScalarGridSpec(
            num_scalar_prefetch=0,
            grid=(B,),
            in_specs=[
                pl.BlockSpec((1, 2 * N, F), lambda b: (b, 0, 0)),
                pl.BlockSpec((1, N, N, E), lambda b: (b, 0, 0, 0)),
                pl.BlockSpec((R, 5 * F), lambda b: (0, 0)),
            ],
            out_specs=pl.BlockSpec((1, 1, F), lambda b: (b, 0, 0)),
        ),
        compiler_params=pltpu.CompilerParams(
            dimension_semantics=("parallel",),
        ),
        cost_estimate=cost,
    )(hadj, edge, big)
    return out.reshape(B, F)
